# Initial kernel scaffold; baseline (speedup 1.0000x reference)
import jax
import jax.numpy as jnp
from jax.experimental import pallas as pl


def kernel(queries, keys, values, adj):
    raise NotImplementedError("write your pallas kernel here")


# speedup vs baseline: 65.4393x; 65.4393x over previous
import functools
import math

import jax
import jax.numpy as jnp
from jax import lax
from jax.experimental import pallas as pl
from jax.experimental.pallas import tpu as pltpu
from jax.experimental.pallas import tpu_sc as plsc

_NC = 2
_NS = 16
_SLAB = 32
_CHUNK = 16384


def _build_count_matrix(dst, src, zeros_slab, l):
    n_edges = dst.shape[0]
    n_workers = _NC * _NS
    rows_per_worker = l // n_workers
    slabs_per_worker = rows_per_worker // _SLAB
    n_chunks = n_edges // _CHUNK
    groups_per_chunk = _CHUNK // 16

    mesh = plsc.VectorSubcoreMesh(core_axis_name="c", subcore_axis_name="s")

    @functools.partial(
        pl.kernel,
        mesh=mesh,
        compiler_params=pltpu.CompilerParams(needs_layout_passes=False),
        out_type=jax.ShapeDtypeStruct((l * l,), jnp.float32),
        scratch_types=[
            pltpu.VMEM((_SLAB * l,), jnp.float32),
            pltpu.VMEM((_CHUNK,), jnp.int32),
            pltpu.VMEM((_CHUNK,), jnp.int32),
        ],
    )
    def build(dst_hbm, src_hbm, zeros_hbm, c_hbm, slab, dst_buf, src_buf):
        wid = lax.axis_index("s") * _NC + lax.axis_index("c")
        ones = jnp.full((16,), 1.0, jnp.float32)
        for si in range(slabs_per_worker):
            row_base = wid * rows_per_worker + si * _SLAB
            pltpu.sync_copy(zeros_hbm, slab)

            def chunk_body(ci, _, row_base=row_base):
                off = ci * _CHUNK
                pltpu.sync_copy(dst_hbm.at[pl.ds(off, _CHUNK)], dst_buf)
                pltpu.sync_copy(src_hbm.at[pl.ds(off, _CHUNK)], src_buf)

                def group_body(g, _):
                    d = dst_buf[pl.ds(g * 16, 16)]
                    s = src_buf[pl.ds(g * 16, 16)]
                    dl = d - row_base
                    mask = (dl >= 0) & (dl < _SLAB)
                    dl = jnp.where(mask, dl, 0)
                    plsc.addupdate_scatter(slab, [dl * l + s], ones, mask=mask)
                    return 0

                lax.fori_loop(0, groups_per_chunk, group_body, 0)
                return 0

            lax.fori_loop(0, n_chunks, chunk_body, 0)
            pltpu.sync_copy(slab, c_hbm.at[pl.ds(row_base * l, _SLAB * l)])

    return build(dst, src, zeros_slab).reshape(l, l)


def _dense_masked_attention(qt, kt, vt, c):
    n, h, l, e = qt.shape
    temp = 1.0 / math.sqrt(e)
    bq = 256
    n_bq = l // bq

    def body(q_ref, k_ref, v_ref, c_ref, o_ref):
        k = k_ref[0, 0]
        v = v_ref[0, 0]

        def chunk(i, _):
            qc = q_ref[0, 0, pl.ds(i * bq, bq), :]
            s = temp * lax.dot_general(
                qc, k, (((1,), (1,)), ((), ())),
                preferred_element_type=jnp.float32)
            cc = c_ref[pl.ds(i * bq, bq), :]
            sm = jnp.where(cc > 0.0, s, -1e30)
            m = jnp.max(sm, axis=1, keepdims=True)
            m = jnp.where(m < -1e29, 0.0, m)
            w = cc * jnp.exp(sm - m)
            denom = jnp.sum(w, axis=1, keepdims=True)
            o = lax.dot_general(
                w, v, (((1,), (0,)), ((), ())),
                preferred_element_type=jnp.float32)
            o_ref[0, 0, pl.ds(i * bq, bq), :] = o / (denom + 1e-16)
            return 0

        lax.fori_loop(0, n_bq, chunk, 0)

    return pl.pallas_call(
        body,
        grid=(n, h),
        in_specs=[
            pl.BlockSpec((1, 1, l, e), lambda i, j: (i, j, 0, 0)),
            pl.BlockSpec((1, 1, l, e), lambda i, j: (i, j, 0, 0)),
            pl.BlockSpec((1, 1, l, e), lambda i, j: (i, j, 0, 0)),
            pl.BlockSpec((l, l), lambda i, j: (0, 0)),
        ],
        out_specs=pl.BlockSpec((1, 1, l, e), lambda i, j: (i, j, 0, 0)),
        out_shape=jax.ShapeDtypeStruct((n, h, l, e), jnp.float32),
    )(qt, kt, vt, c)


def kernel(queries, keys, values, adj):
    n, l, h, e = queries.shape
    dst = adj[0]
    src = adj[1]
    zeros_slab = jnp.zeros((_SLAB * l,), jnp.float32)
    c = _build_count_matrix(dst, src, zeros_slab, l)
    qt = jnp.transpose(queries, (0, 2, 1, 3))
    kt = jnp.transpose(keys, (0, 2, 1, 3))
    vt = jnp.transpose(values, (0, 2, 1, 3))
    out = _dense_masked_attention(qt, kt, vt, c)
    return jnp.transpose(out, (0, 2, 1, 3))
